# baseline (speedup 1.0000x reference)
import functools

import jax
import jax.numpy as jnp
from jax import lax
from jax.experimental import pallas as pl
from jax.experimental.pallas import tpu as pltpu
from jax.experimental.pallas import tpu_sc as plsc

_NUM_CLASSES = 100000
_FEAT = 64
_BATCH = 16384
_NC = 2
_NS = 16
_L = 16
_NW = _NC * _NS
_ROWS_W = _BATCH // _NW
_ROWS_S = _BATCH // _NS
_HIST_PAD = 100352
_HCHUNK = _HIST_PAD // _NS


def _hist_body(lbl_hbm, zeros_hbm, ones_hbm, w_hbm,
               lbl_v, lblf_v, ones_v, cnt_v, wv_v, hist, ):
    c = lax.axis_index("c")
    s = lax.axis_index("s")
    base = s * _ROWS_S + c * _ROWS_W
    pltpu.sync_copy(lbl_hbm.at[pl.ds(s * _ROWS_S, _ROWS_S)], lblf_v)
    pltpu.sync_copy(ones_hbm, ones_v)
    pltpu.sync_copy(zeros_hbm.at[pl.ds(s * _HCHUNK, _HCHUNK)],
                    hist.at[pl.ds(s * _HCHUNK, _HCHUNK)])
    for i in range(_ROWS_S // _L):
        lbl_v[i // 8, pl.ds((i % 8) * _L, _L)] = lblf_v[pl.ds(i * _L, _L)]
    plsc.subcore_barrier()
    for j in range(8):
        pltpu.sync_copy(ones_v, hist.at[lbl_v.at[j]], add=True)
    plsc.subcore_barrier()
    for j in range(4):
        pltpu.sync_copy(hist.at[lbl_v.at[c * 4 + j]],
                        cnt_v.at[pl.ds(j * 128, 128)])
    for i in range(_ROWS_W // _L):
        wv_v[pl.ds(i * _L, _L)] = 0.5 / (cnt_v[pl.ds(i * _L, _L)] + 1.0)
    pltpu.sync_copy(wv_v, w_hbm.at[pl.ds(pl.multiple_of(base, 8), _ROWS_W)])


_hist_call = functools.partial(
    pl.kernel,
    mesh=plsc.VectorSubcoreMesh(core_axis_name="c", subcore_axis_name="s"),
    out_type=jax.ShapeDtypeStruct((_BATCH,), jnp.float32),
    compiler_params=pltpu.CompilerParams(use_tc_tiling_on_sc=True),
    scratch_types=[
        pltpu.VMEM((8, 128), jnp.int32),
        pltpu.VMEM((_ROWS_S,), jnp.int32),
        pltpu.VMEM((128,), jnp.float32),
        pltpu.VMEM((_ROWS_W,), jnp.float32),
        pltpu.VMEM((_ROWS_W,), jnp.float32),
        pltpu.VMEM_SHARED((_HIST_PAD,), jnp.float32),
    ],
)(_hist_body)


def _dist_body(xp_hbm, lbl_hbm, cp_hbm, w_hbm, out_hbm,
               lblf_v, tid_v, xv, cv, wv, acc_v, sem_c, sem_x):
    c = lax.axis_index("c")
    s = lax.axis_index("s")
    base = s * _ROWS_S + c * _ROWS_W
    pltpu.sync_copy(lbl_hbm.at[pl.ds(pl.multiple_of(base, 8), _ROWS_W)],
                    lblf_v)
    cp_x = pltpu.async_copy(
        xp_hbm.at[pl.ds(pl.multiple_of(base // 2, 256), _ROWS_W // 2)], xv,
        sem_x)
    for i in range(_ROWS_W // _L):
        tid_v[i // 8, pl.ds((i % 8) * _L, _L)] = lblf_v[pl.ds(i * _L, _L)]
    cps = []
    for j in range(4):
        cps.append(pltpu.async_copy(
            cp_hbm.at[tid_v.at[j]], cv.at[pl.ds(j * 128, 128)], sem_c))
    pltpu.sync_copy(w_hbm.at[pl.ds(pl.multiple_of(base, 8), _ROWS_W)], wv)
    cp_x.wait()

    def dbody(g, acc):
        wblk = wv[pl.ds(g * _L, _L)]
        for j in range(_L):
            r = g * _L + j
            xrow = g * 8 + (j // 2)
            xc = (j % 2) * _FEAT
            ssq = None
            for k in range(_FEAT // _L):
                d = (xv[xrow, pl.ds(xc + k * _L, _L)]
                     - cv[r, pl.ds(k * _L, _L)])
                ssq = d * d if ssq is None else ssq + d * d
            acc = acc + lax.broadcast(wblk[j], (_L,)) * ssq
        return acc

    acc = jnp.zeros((_L,), jnp.float32)
    for j in range(4):
        cps[j].wait()
        acc = lax.fori_loop(j * 8, (j + 1) * 8, dbody, acc)
    acc_v[...] = acc
    pltpu.sync_copy(acc_v, out_hbm.at[s * _NC + c])


_dist_call = functools.partial(
    pl.kernel,
    mesh=plsc.VectorSubcoreMesh(core_axis_name="c", subcore_axis_name="s"),
    out_type=jax.ShapeDtypeStruct((_NW, _L), jnp.float32),
    compiler_params=pltpu.CompilerParams(use_tc_tiling_on_sc=True),
    scratch_types=[
        pltpu.VMEM((_ROWS_W,), jnp.int32),
        pltpu.VMEM((4, 128), jnp.int32),
        pltpu.VMEM((_ROWS_W // 2, 128), jnp.float32),
        pltpu.VMEM((_ROWS_W, 128), jnp.float32),
        pltpu.VMEM((_ROWS_W,), jnp.float32),
        pltpu.VMEM((_L,), jnp.float32),
        pltpu.SemaphoreType.DMA,
        pltpu.SemaphoreType.DMA,
    ],
)(_dist_body)


def kernel(x, labels, center):
    lbl1d = labels.astype(jnp.int32)
    xp = x.reshape(_BATCH // 2, 128)
    cp = jnp.pad(center, ((0, 0), (0, 64)))
    zeros = jnp.zeros((_HIST_PAD,), jnp.float32)
    ones = jnp.ones((128,), jnp.float32)
    w = _hist_call(lbl1d, zeros, ones)
    out = _dist_call(xp, lbl1d, cp, w)
    return jnp.sum(out)


# speedup vs baseline: 1.0197x; 1.0197x over previous
import functools

import jax
import jax.numpy as jnp
from jax import lax
from jax.experimental import pallas as pl
from jax.experimental.pallas import tpu as pltpu
from jax.experimental.pallas import tpu_sc as plsc

_NUM_CLASSES = 100000
_FEAT = 64
_BATCH = 16384
_NC = 2
_NS = 16
_L = 16
_NW = _NC * _NS
_ROWS_W = _BATCH // _NW
_ROWS_S = _BATCH // _NS
_HIST_PAD = 100352
_HCHUNK = _HIST_PAD // _NS


def _hist_body(lbl_hbm, zeros_hbm, ones_hbm, w_hbm,
               lbl_v, lblf_v, ones_v, cnt_v, wv_v, hist, ):
    c = lax.axis_index("c")
    s = lax.axis_index("s")
    base = s * _ROWS_S + c * _ROWS_W
    pltpu.sync_copy(lbl_hbm.at[pl.ds(s * _ROWS_S, _ROWS_S)], lblf_v)
    pltpu.sync_copy(ones_hbm, ones_v)
    pltpu.sync_copy(zeros_hbm.at[pl.ds(s * _HCHUNK, _HCHUNK)],
                    hist.at[pl.ds(s * _HCHUNK, _HCHUNK)])
    for i in range(_ROWS_S // _L):
        lbl_v[i // 8, pl.ds((i % 8) * _L, _L)] = lblf_v[pl.ds(i * _L, _L)]
    plsc.subcore_barrier()
    for j in range(8):
        pltpu.sync_copy(ones_v, hist.at[lbl_v.at[j]], add=True)
    plsc.subcore_barrier()
    for j in range(4):
        pltpu.sync_copy(hist.at[lbl_v.at[c * 4 + j]],
                        cnt_v.at[pl.ds(j * 128, 128)])
    for i in range(_ROWS_W // _L):
        wv_v[pl.ds(i * _L, _L)] = 0.5 / (cnt_v[pl.ds(i * _L, _L)] + 1.0)
    pltpu.sync_copy(wv_v, w_hbm.at[pl.ds(pl.multiple_of(base, 8), _ROWS_W)])


_hist_call = functools.partial(
    pl.kernel,
    mesh=plsc.VectorSubcoreMesh(core_axis_name="c", subcore_axis_name="s"),
    out_type=jax.ShapeDtypeStruct((_BATCH,), jnp.float32),
    compiler_params=pltpu.CompilerParams(use_tc_tiling_on_sc=True),
    scratch_types=[
        pltpu.VMEM((8, 128), jnp.int32),
        pltpu.VMEM((_ROWS_S,), jnp.int32),
        pltpu.VMEM((128,), jnp.float32),
        pltpu.VMEM((_ROWS_W,), jnp.float32),
        pltpu.VMEM((_ROWS_W,), jnp.float32),
        pltpu.VMEM_SHARED((_HIST_PAD,), jnp.float32),
    ],
)(_hist_body)


def _dist_body(xp_hbm, lbl_hbm, cp_hbm, w_hbm, out_hbm,
               lblf_v, tid_v, xv, cv, wv, acc_v, sem_c, sem_x):
    c = lax.axis_index("c")
    s = lax.axis_index("s")
    base = s * _ROWS_S + c * _ROWS_W
    pltpu.sync_copy(lbl_hbm.at[pl.ds(pl.multiple_of(base, 8), _ROWS_W)],
                    lblf_v)
    cp_x = pltpu.async_copy(
        xp_hbm.at[pl.ds(pl.multiple_of(base // 2, 256), _ROWS_W // 2)], xv,
        sem_x)
    for i in range(_ROWS_W // _L):
        tid_v[i // 8, pl.ds((i % 8) * _L, _L)] = lblf_v[pl.ds(i * _L, _L)]
    cps = []
    for j in range(4):
        cps.append(pltpu.async_copy(
            cp_hbm.at[tid_v.at[j]], cv.at[pl.ds(j * 128, 128)], sem_c))
    pltpu.sync_copy(w_hbm.at[pl.ds(pl.multiple_of(base, 8), _ROWS_W)], wv)
    cp_x.wait()

    def dbody(g, acc):
        wblk = wv[pl.ds(g * _L, _L)]
        for j in range(_L):
            r = g * _L + j
            xrow = g * 8 + (j // 2)
            xc = (j % 2) * _FEAT
            ssq = None
            for k in range(_FEAT // _L):
                d = (xv[xrow, pl.ds(xc + k * _L, _L)]
                     - cv[r, pl.ds(k * _L, _L)])
                ssq = d * d if ssq is None else ssq + d * d
            acc = acc + lax.broadcast(wblk[j], (_L,)) * ssq
        return acc

    for cp in cps:
        cp.wait()
    acc = lax.fori_loop(0, _ROWS_W // _L, dbody,
                        jnp.zeros((_L,), jnp.float32))
    acc_v[...] = acc
    pltpu.sync_copy(acc_v, out_hbm.at[s * _NC + c])


_dist_call = functools.partial(
    pl.kernel,
    mesh=plsc.VectorSubcoreMesh(core_axis_name="c", subcore_axis_name="s"),
    out_type=jax.ShapeDtypeStruct((_NW, _L), jnp.float32),
    compiler_params=pltpu.CompilerParams(use_tc_tiling_on_sc=True),
    scratch_types=[
        pltpu.VMEM((_ROWS_W,), jnp.int32),
        pltpu.VMEM((4, 128), jnp.int32),
        pltpu.VMEM((_ROWS_W // 2, 128), jnp.float32),
        pltpu.VMEM((_ROWS_W, 128), jnp.float32),
        pltpu.VMEM((_ROWS_W,), jnp.float32),
        pltpu.VMEM((_L,), jnp.float32),
        pltpu.SemaphoreType.DMA,
        pltpu.SemaphoreType.DMA,
    ],
)(_dist_body)


def kernel(x, labels, center):
    lbl1d = labels.astype(jnp.int32)
    xp = x.reshape(_BATCH // 2, 128)
    cp = jnp.pad(center, ((0, 0), (0, 64)))
    zeros = jnp.zeros((_HIST_PAD,), jnp.float32)
    ones = jnp.ones((128,), jnp.float32)
    w = _hist_call(lbl1d, zeros, ones)
    out = _dist_call(xp, lbl1d, cp, w)
    return jnp.sum(out)
